# baseline (speedup 1.0000x reference)
import functools
import math

import jax
import jax.numpy as jnp
from jax import lax
from jax.experimental import pallas as pl
from jax.experimental.pallas import tpu as pltpu
from jax.experimental.pallas import tpu_sc as plsc

B = 16384
S_DIM = 32
T_DIM = 32
PI_REF = 3.1415926235897933
GAMMA = 12.0
EMB_RANGE = GAMMA / float(S_DIM + T_DIM)
INV_2SCALE = PI_REF / (2.0 * EMB_RANGE)

NW = 32
QPW = B // NW
CH = 32
NCHUNK = QPW // CH

_TWO_PI = 6.283185307179586
_INV_2PI = 0.15915494309189535
_C3 = -0.1666666587584901
_C5 = 0.00833332023467762
_C7 = -0.00019840491560017788
_C9 = 2.7535159818767513e-06
_C11 = -2.472396353305536e-08
_C13 = 1.3601221017511822e-10


def _sin(x):
    k = x * _INV_2PI
    k = k + 0.5 * jnp.sign(k)
    kf = lax.convert_element_type(lax.convert_element_type(k, jnp.int32), jnp.float32)
    r = x - kf * _TWO_PI
    r2 = r * r
    p = _C13
    p = p * r2 + _C11
    p = p * r2 + _C9
    p = p * r2 + _C7
    p = p * r2 + _C5
    p = p * r2 + _C3
    return r + r * (r2 * p)


def _sinc(z):
    s = z * math.pi
    return jnp.where(z == 0.0, jnp.float32(1.0), _sin(s) / s)


def _sqrt(a):
    i = lax.bitcast_convert_type(a, jnp.int32)
    i = 0x5F3759DF - lax.shift_right_logical(i, 1)
    y = lax.bitcast_convert_type(i, jnp.float32)
    for _ in range(3):
        y = y * (1.5 - 0.5 * a * y * y)
    return jnp.where(a > 0.0, a * y, jnp.float32(0.0))


def _make_kernel():
    mesh = plsc.VectorSubcoreMesh(core_axis_name="c", subcore_axis_name="s")
    f32 = jnp.float32
    i32 = jnp.int32

    scratch = (
        [pltpu.VMEM((QPW,), i32)] * 6
        + [pltpu.VMEM((QPW,), f32)] * 3
        + [pltpu.VMEM((CH, 128), f32)] * 25
        + [pltpu.VMEM((CH * 16,), f32)] * 2
        + [pltpu.VMEM((QPW,), f32)]
        + [pltpu.SemaphoreType.DMA]
    )

    @functools.partial(
        pl.kernel,
        out_type=jax.ShapeDtypeStruct((B,), f32),
        mesh=mesh,
        scratch_types=scratch,
        compiler_params=pltpu.CompilerParams(needs_layout_passes=False),
    )
    def sc_kernel(heads, rels, tails, years, months, days,
                  ent_h, ent_t, rel_f, rel_i, rel_j,
                  m_fh, m_ft, m_ph, m_pt, m_ah, m_at,
                  d_fh, d_ft, d_ph, d_pt, d_ah, d_at,
                  y_fh, y_ft, y_ph, y_pt, y_ah, y_at,
                  out,
                  hv, tv, rv, hv4, tv4, rv2, yv, mv, dv,
                  g_eh_h, g_et_h, g_eh_t, g_et_t,
                  g_yfh, g_yph, g_yah, g_mfh, g_mph, g_mah,
                  g_dfh, g_dph, g_dah,
                  g_yft, g_ypt, g_yat, g_mft, g_mpt, g_mat,
                  g_dft, g_dpt, g_dat,
                  g_r1, g_r2, g_r3,
                  psb, msb, outv, sem):
        wid = lax.axis_index("s") * 2 + lax.axis_index("c")
        base = wid * QPW

        pltpu.sync_copy(heads.at[pl.ds(base, QPW)], hv)
        pltpu.sync_copy(tails.at[pl.ds(base, QPW)], tv)
        pltpu.sync_copy(rels.at[pl.ds(base, QPW)], rv)
        pltpu.sync_copy(years.at[pl.ds(base, QPW)], yv)
        pltpu.sync_copy(months.at[pl.ds(base, QPW)], mv)
        pltpu.sync_copy(days.at[pl.ds(base, QPW)], dv)

        def tbody(i, carry):
            s = pl.multiple_of(i * 16, 16)
            sl = pl.ds(s, 16)
            hv4[sl] = lax.shift_right_logical(hv[sl], 2)
            tv4[sl] = lax.shift_right_logical(tv[sl], 2)
            rv2[sl] = lax.shift_right_logical(rv[sl], 1)
            yv[sl] = yv[sl] - 2010.0
            mv[sl] = mv[sl] * (1.0 / 6.0) - 1.0
            dv[sl] = dv[sl] * 0.0625 - 1.0
            return carry
        lax.fori_loop(0, QPW // 16, tbody, 0)

        head_bufs = [g_eh_h, g_et_h, g_yfh, g_yph, g_yah,
                     g_mfh, g_mph, g_mah, g_dfh, g_dph, g_dah]
        head_tabs = [ent_h, ent_t, y_fh, y_ph, y_ah,
                     m_fh, m_ph, m_ah, d_fh, d_ph, d_ah]
        tail_bufs = [g_eh_t, g_et_t, g_yft, g_ypt, g_yat,
                     g_mft, g_mpt, g_mat, g_dft, g_dpt, g_dat]
        tail_tabs = [ent_h, ent_t, y_ft, y_pt, y_at,
                     m_ft, m_pt, m_at, d_ft, d_pt, d_at]
        rel_bufs = [g_r1, g_r2, g_r3]
        rel_tabs = [rel_f, rel_i, rel_j]

        iota = lax.iota(i32, 16)

        def cbody(c, carry0):
            co = c * CH
            handles = []
            hidx = hv4.at[pl.ds(co, CH)]
            tidx = tv4.at[pl.ds(co, CH)]
            ridx = rv2.at[pl.ds(co, CH)]
            for tab, buf in zip(head_tabs, head_bufs):
                handles.append(pltpu.async_copy(tab.at[hidx], buf, sem))
            for tab, buf in zip(tail_tabs, tail_bufs):
                handles.append(pltpu.async_copy(tab.at[tidx], buf, sem))
            for tab, buf in zip(rel_tabs, rel_bufs):
                handles.append(pltpu.async_copy(tab.at[ridx], buf, sem))
            for h in handles:
                h.wait()

            def qbody(q, carry):
                qs = jnp.full((16,), q, i32)
                gq = jnp.full((16,), co + q, i32)
                eh = plsc.load_gather(hv, [gq])
                et = plsc.load_gather(tv, [gq])
                er = plsc.load_gather(rv, [gq])
                csh = lax.shift_left(eh & 3, 5) + iota
                cst = lax.shift_left(et & 3, 5) + iota
                csr = lax.shift_left(er & 1, 6) + iota
                ty = plsc.load_gather(yv, [gq])
                tm = plsc.load_gather(mv, [gq])
                td = plsc.load_gather(dv, [gq])
                ps = jnp.zeros((16,), f32)
                ms = jnp.zeros((16,), f32)
                for c2 in (0, 16):
                    ch = csh + c2
                    ct = cst + c2
                    cr = csr + c2
                    cr2 = cr + 32

                    def lg(buf, col):
                        return plsc.load_gather(buf, [qs, col])

                    th = (lg(g_yah, ch) * _sinc(lg(g_yfh, ch) * ty + lg(g_yph, ch))
                          + lg(g_mah, ch) * _sinc(lg(g_mfh, ch) * tm + lg(g_mph, ch))
                          + lg(g_dah, ch) * _sinc(lg(g_dfh, ch) * td + lg(g_dph, ch)))
                    tt = (lg(g_yat, ct) * _sinc(lg(g_yft, ct) * ty + lg(g_ypt, ct))
                          + lg(g_mat, ct) * _sinc(lg(g_mft, ct) * tm + lg(g_mpt, ct))
                          + lg(g_dat, ct) * _sinc(lg(g_dft, ct) * td + lg(g_dpt, ct)))
                    phase1 = (lg(g_eh_h, ch) + lg(g_r1, cr) - lg(g_et_t, ct)) * INV_2SCALE
                    phase2 = (th + lg(g_r1, cr2) - tt) * INV_2SCALE
                    ps = ps + jnp.abs(_sin(phase1)) + jnp.abs(_sin(phase2))
                    moda = jnp.abs(lg(g_r2, cr))
                    biasa = jnp.maximum(jnp.minimum(lg(g_r3, cr), 1.0), -moda)
                    rsc1 = lg(g_eh_t, ct) * (moda + biasa) - lg(g_et_h, ch) * (1.0 - biasa)
                    modb = jnp.abs(lg(g_r2, cr2))
                    biasb = jnp.maximum(jnp.minimum(lg(g_r3, cr2), 1.0), -modb)
                    rsc2 = th * (modb + biasb) - tt * (1.0 - biasb)
                    ms = ms + rsc1 * rsc1 + rsc2 * rsc2
                qo = pl.multiple_of(q * 16, 16)
                psb[pl.ds(qo, 16)] = ps
                msb[pl.ds(qo, 16)] = ms
                return carry
            lax.fori_loop(0, CH, qbody, 0)

            def gbody(g, carry):
                def kbody(k, accs):
                    ap, am = accs
                    idx = (iota + g * 16) * 16 + k
                    ap = ap + plsc.load_gather(psb, [idx])
                    am = am + plsc.load_gather(msb, [idx])
                    return (ap, am)
                ap, am = lax.fori_loop(
                    0, 16, kbody,
                    (jnp.zeros((16,), f32), jnp.zeros((16,), f32)))
                res = GAMMA - (0.5 * ap + _sqrt(am))
                outv[pl.ds(co + g * 16, 16)] = res
                return carry
            lax.fori_loop(0, CH // 16, gbody, 0)
            return carry0

        lax.fori_loop(0, NCHUNK, cbody, 0)

        pltpu.sync_copy(outv, out.at[pl.ds(base, QPW)])

    return sc_kernel


_sc_kernel_cache = []


def _get_sc_kernel():
    if not _sc_kernel_cache:
        _sc_kernel_cache.append(_make_kernel())
    return _sc_kernel_cache[0]


@jax.jit
def kernel(heads, rels, tails, years, months, days,
           ent_embs_h, ent_embs_t, rel_embs_f, rel_embs_i, rel_embs_j,
           m_freq_h, m_freq_t, m_phi_h, m_phi_t, m_amps_h, m_amps_t,
           d_freq_h, d_freq_t, d_phi_h, d_phi_t, d_amps_h, d_amps_t,
           y_freq_h, y_freq_t, y_phi_h, y_phi_t, y_amps_h, y_amps_t):
    def r4(t):
        return jnp.reshape(t, (t.shape[0] // 4, 128))

    def r2(t):
        return jnp.reshape(t, (t.shape[0] // 2, 128))

    return _get_sc_kernel()(
        heads, rels, tails, years, months, days,
        r4(ent_embs_h), r4(ent_embs_t),
        r2(rel_embs_f), r2(rel_embs_i), r2(rel_embs_j),
        r4(m_freq_h), r4(m_freq_t), r4(m_phi_h), r4(m_phi_t),
        r4(m_amps_h), r4(m_amps_t),
        r4(d_freq_h), r4(d_freq_t), r4(d_phi_h), r4(d_phi_t),
        r4(d_amps_h), r4(d_amps_t),
        r4(y_freq_h), r4(y_freq_t), r4(y_phi_h), r4(y_phi_t),
        r4(y_amps_h), r4(y_amps_t))


# speedup vs baseline: 1.0903x; 1.0903x over previous
import functools
import math

import jax
import jax.numpy as jnp
from jax import lax
from jax.experimental import pallas as pl
from jax.experimental.pallas import tpu as pltpu
from jax.experimental.pallas import tpu_sc as plsc

B = 16384
S_DIM = 32
T_DIM = 32
PI_REF = 3.1415926235897933
GAMMA = 12.0
EMB_RANGE = GAMMA / float(S_DIM + T_DIM)
INV_2SCALE = PI_REF / (2.0 * EMB_RANGE)

NW = 32
QPW = B // NW
CH = 64
NCHUNK = QPW // CH

_TWO_PI = 6.283185307179586
_INV_2PI = 0.15915494309189535
_C3 = -0.1666666587584901
_C5 = 0.00833332023467762
_C7 = -0.00019840491560017788
_C9 = 2.7535159818767513e-06
_C11 = -2.472396353305536e-08
_C13 = 1.3601221017511822e-10


def _sin(x):
    k = x * _INV_2PI
    k = k + 0.5 * jnp.sign(k)
    kf = lax.convert_element_type(lax.convert_element_type(k, jnp.int32), jnp.float32)
    r = x - kf * _TWO_PI
    r2 = r * r
    p = _C13
    p = p * r2 + _C11
    p = p * r2 + _C9
    p = p * r2 + _C7
    p = p * r2 + _C5
    p = p * r2 + _C3
    return r + r * (r2 * p)


def _sinc(z):
    s = z * math.pi
    return jnp.where(z == 0.0, jnp.float32(1.0), _sin(s) / s)


def _sqrt(a):
    i = lax.bitcast_convert_type(a, jnp.int32)
    i = 0x5F3759DF - lax.shift_right_logical(i, 1)
    y = lax.bitcast_convert_type(i, jnp.float32)
    for _ in range(3):
        y = y * (1.5 - 0.5 * a * y * y)
    return jnp.where(a > 0.0, a * y, jnp.float32(0.0))


def _make_kernel():
    mesh = plsc.VectorSubcoreMesh(core_axis_name="c", subcore_axis_name="s")
    f32 = jnp.float32
    i32 = jnp.int32

    scratch = (
        [pltpu.VMEM((QPW,), i32)] * 3
        + [pltpu.VMEM((QPW,), f32)] * 3
        + [pltpu.VMEM((2 * CH, S_DIM), f32)] * 22
        + [pltpu.VMEM((2 * CH, 2 * S_DIM), f32)] * 3
        + [pltpu.VMEM((CH * 16,), f32)] * 2
        + [pltpu.VMEM((QPW,), f32)]
        + [pltpu.SemaphoreType.DMA]
    )

    @functools.partial(
        pl.kernel,
        out_type=jax.ShapeDtypeStruct((B,), f32),
        mesh=mesh,
        scratch_types=scratch,
        compiler_params=pltpu.CompilerParams(
            needs_layout_passes=False, use_tc_tiling_on_sc=False),
    )
    def sc_kernel(heads, rels, tails, years, months, days,
                  ent_h, ent_t, rel_f, rel_i, rel_j,
                  m_fh, m_ft, m_ph, m_pt, m_ah, m_at,
                  d_fh, d_ft, d_ph, d_pt, d_ah, d_at,
                  y_fh, y_ft, y_ph, y_pt, y_ah, y_at,
                  out,
                  hv, tv, rv, yv, mv, dv,
                  g_eh_h, g_et_h, g_eh_t, g_et_t,
                  g_yfh, g_yph, g_yah, g_mfh, g_mph, g_mah,
                  g_dfh, g_dph, g_dah,
                  g_yft, g_ypt, g_yat, g_mft, g_mpt, g_mat,
                  g_dft, g_dpt, g_dat,
                  g_r1, g_r2, g_r3,
                  psb, msb, outv, sem):
        wid = lax.axis_index("s") * 2 + lax.axis_index("c")
        base = wid * QPW

        pltpu.sync_copy(heads.at[pl.ds(base, QPW)], hv)
        pltpu.sync_copy(tails.at[pl.ds(base, QPW)], tv)
        pltpu.sync_copy(rels.at[pl.ds(base, QPW)], rv)
        pltpu.sync_copy(years.at[pl.ds(base, QPW)], yv)
        pltpu.sync_copy(months.at[pl.ds(base, QPW)], mv)
        pltpu.sync_copy(days.at[pl.ds(base, QPW)], dv)

        def tbody(i, carry):
            s = pl.multiple_of(i * 16, 16)
            sl = pl.ds(s, 16)
            yv[sl] = yv[sl] - 2010.0
            mv[sl] = mv[sl] * (1.0 / 6.0) - 1.0
            dv[sl] = dv[sl] * 0.0625 - 1.0
            return carry
        lax.fori_loop(0, QPW // 16, tbody, 0)

        head_tabs = [(ent_h, g_eh_h), (ent_t, g_et_h),
                     (y_fh, g_yfh), (y_ph, g_yph), (y_ah, g_yah),
                     (m_fh, g_mfh), (m_ph, g_mph), (m_ah, g_mah),
                     (d_fh, g_dfh), (d_ph, g_dph), (d_ah, g_dah)]
        tail_tabs = [(ent_h, g_eh_t), (ent_t, g_et_t),
                     (y_ft, g_yft), (y_pt, g_ypt), (y_at, g_yat),
                     (m_ft, g_mft), (m_pt, g_mpt), (m_at, g_mat),
                     (d_ft, g_dft), (d_pt, g_dpt), (d_at, g_dat)]
        rel_tabs = [(rel_f, g_r1), (rel_i, g_r2), (rel_j, g_r3)]
        all_tabs = (head_tabs, tail_tabs, rel_tabs)

        def issue(c, par):
            co = c * CH
            off = par * CH
            hidx = hv.at[pl.ds(co, CH)]
            tidx = tv.at[pl.ds(co, CH)]
            ridx = rv.at[pl.ds(co, CH)]
            for idx, tabs in zip((hidx, tidx, ridx), all_tabs):
                for tab, buf in tabs:
                    pltpu.async_copy(tab.at[idx], buf.at[pl.ds(off, CH)], sem)

        def drain(par):
            off = par * CH
            for idx_src, tabs in zip((hv, tv, rv), all_tabs):
                for tab, buf in tabs:
                    pltpu.make_async_copy(
                        tab.at[idx_src.at[pl.ds(0, CH)]],
                        buf.at[pl.ds(off, CH)], sem).wait()

        issue(0, 0)

        def cbody(c, carry0):
            co = c * CH
            par = lax.rem(c, 2)
            poff = par * CH

            @pl.when(c + 1 < NCHUNK)
            def _():
                issue(c + 1, 1 - par)

            drain(par)

            def qbody(q, carry):
                gq = jnp.full((16,), co + q, i32)
                row = poff + q
                ty = plsc.load_gather(yv, [gq])
                tm = plsc.load_gather(mv, [gq])
                td = plsc.load_gather(dv, [gq])
                ps = jnp.zeros((16,), f32)
                ms = jnp.zeros((16,), f32)
                for c2 in (0, 16):
                    dsl = pl.ds(c2, 16)
                    dsl2 = pl.ds(32 + c2, 16)
                    th = (g_yah[row, dsl] * _sinc(g_yfh[row, dsl] * ty + g_yph[row, dsl])
                          + g_mah[row, dsl] * _sinc(g_mfh[row, dsl] * tm + g_mph[row, dsl])
                          + g_dah[row, dsl] * _sinc(g_dfh[row, dsl] * td + g_dph[row, dsl]))
                    tt = (g_yat[row, dsl] * _sinc(g_yft[row, dsl] * ty + g_ypt[row, dsl])
                          + g_mat[row, dsl] * _sinc(g_mft[row, dsl] * tm + g_mpt[row, dsl])
                          + g_dat[row, dsl] * _sinc(g_dft[row, dsl] * td + g_dpt[row, dsl]))
                    phase1 = (g_eh_h[row, dsl] + g_r1[row, dsl] - g_et_t[row, dsl]) * INV_2SCALE
                    phase2 = (th + g_r1[row, dsl2] - tt) * INV_2SCALE
                    ps = ps + jnp.abs(_sin(phase1)) + jnp.abs(_sin(phase2))
                    moda = jnp.abs(g_r2[row, dsl])
                    biasa = jnp.maximum(jnp.minimum(g_r3[row, dsl], 1.0), -moda)
                    rsc1 = g_eh_t[row, dsl] * (moda + biasa) - g_et_h[row, dsl] * (1.0 - biasa)
                    modb = jnp.abs(g_r2[row, dsl2])
                    biasb = jnp.maximum(jnp.minimum(g_r3[row, dsl2], 1.0), -modb)
                    rsc2 = th * (modb + biasb) - tt * (1.0 - biasb)
                    ms = ms + rsc1 * rsc1 + rsc2 * rsc2
                qo = pl.multiple_of(q * 16, 16)
                psb[pl.ds(qo, 16)] = ps
                msb[pl.ds(qo, 16)] = ms
                return carry
            lax.fori_loop(0, CH, qbody, 0)

            iota = lax.iota(i32, 16)

            def gbody(g, carry):
                def kbody(k, accs):
                    ap, am = accs
                    idx = (iota + g * 16) * 16 + k
                    ap = ap + plsc.load_gather(psb, [idx])
                    am = am + plsc.load_gather(msb, [idx])
                    return (ap, am)
                ap, am = lax.fori_loop(
                    0, 16, kbody,
                    (jnp.zeros((16,), f32), jnp.zeros((16,), f32)))
                res = GAMMA - (0.5 * ap + _sqrt(am))
                outv[pl.ds(co + g * 16, 16)] = res
                return carry
            lax.fori_loop(0, CH // 16, gbody, 0)
            return carry0

        lax.fori_loop(0, NCHUNK, cbody, 0)

        pltpu.sync_copy(outv, out.at[pl.ds(base, QPW)])

    return sc_kernel


_sc_kernel_cache = []


def _get_sc_kernel():
    if not _sc_kernel_cache:
        _sc_kernel_cache.append(_make_kernel())
    return _sc_kernel_cache[0]


@jax.jit
def kernel(heads, rels, tails, years, months, days,
           ent_embs_h, ent_embs_t, rel_embs_f, rel_embs_i, rel_embs_j,
           m_freq_h, m_freq_t, m_phi_h, m_phi_t, m_amps_h, m_amps_t,
           d_freq_h, d_freq_t, d_phi_h, d_phi_t, d_amps_h, d_amps_t,
           y_freq_h, y_freq_t, y_phi_h, y_phi_t, y_amps_h, y_amps_t):
    return _get_sc_kernel()(
        heads, rels, tails, years, months, days,
        ent_embs_h, ent_embs_t, rel_embs_f, rel_embs_i, rel_embs_j,
        m_freq_h, m_freq_t, m_phi_h, m_phi_t, m_amps_h, m_amps_t,
        d_freq_h, d_freq_t, d_phi_h, d_phi_t, d_amps_h, d_amps_t,
        y_freq_h, y_freq_t, y_phi_h, y_phi_t, y_amps_h, y_amps_t)
